# baseline (speedup 1.0000x reference)
import functools

import jax
import jax.numpy as jnp
from jax import lax
from jax.experimental import pallas as pl
from jax.experimental.pallas import tpu as pltpu
from jax.experimental.pallas import tpu_sc as plsc

N = 10000
E = 320000
D = 128
H = 128
C = 10
G = 64

NC = 2
NS = 16
NW = NC * NS
K = 128
NCHUNK = 80
E_PAD = NW * K * NCHUNK
N_PAD = 10240
RPT_A = 632
RPT_B = N - (NS - 1) * RPT_A

_mesh = plsc.VectorSubcoreMesh(core_axis_name="c", subcore_axis_name="s")


@functools.partial(
    pl.kernel,
    out_type=jax.ShapeDtypeStruct((NW * N_PAD,), jnp.float32),
    mesh=_mesh,
    scratch_types=[
        pltpu.VMEM((NCHUNK, K), jnp.int32),
        pltpu.VMEM((NCHUNK, K), jnp.float32),
        pltpu.VMEM((N_PAD,), jnp.float32),
    ],
    compiler_params=pltpu.CompilerParams(needs_layout_passes=False),
)
def _sc_wdeg(dstg, ewg, out, dst_v, ew_v, hist):
    core = lax.axis_index("c")
    sid = lax.axis_index("s")
    wid = core * NS + sid

    pltpu.sync_copy(dstg.at[wid], dst_v)
    pltpu.sync_copy(ewg.at[wid], ew_v)

    zv = jnp.zeros((NS,), jnp.float32)

    def zero_body(i, carry):
        hist[pl.ds(i * NS, NS)] = zv
        return carry
    lax.fori_loop(0, N_PAD // NS, zero_body, 0)

    def chunk_body(j, carry):
        def group_body(g, c2):
            idx = dst_v[j, pl.ds(g * NS, NS)]
            w = ew_v[j, pl.ds(g * NS, NS)]
            plsc.addupdate_scatter(hist, [idx], w)
            return c2
        lax.fori_loop(0, K // NS, group_body, 0)
        return carry
    lax.fori_loop(0, NCHUNK, chunk_body, 0)

    pltpu.sync_copy(hist, out.at[pl.ds(wid * N_PAD, N_PAD)])


@functools.partial(
    pl.kernel,
    out_type=jax.ShapeDtypeStruct((NC, N, H), jnp.float32),
    mesh=_mesh,
    scratch_types=[
        pltpu.VMEM((4, K), jnp.int32),
        pltpu.VMEM((4, K), jnp.int32),
        pltpu.VMEM((NCHUNK, K // 2), jnp.int32),
        pltpu.VMEM((K, H // 2), jnp.int32),
        pltpu.VMEM((K, H), jnp.float32),
        pltpu.VMEM((K, H), jnp.float32),
        pltpu.VMEM_SHARED((N, H), jnp.float32),
        pltpu.SemaphoreType.DMA,
        pltpu.SemaphoreType.DMA,
        pltpu.SemaphoreType.DMA,
        pltpu.SemaphoreType.DMA,
        pltpu.SemaphoreType.DMA,
        pltpu.SemaphoreType.DMA,
        pltpu.SemaphoreType.DMA,
    ],
    compiler_params=pltpu.CompilerParams(needs_layout_passes=False,
                                         use_tc_tiling_on_sc=False),
)
def _sc_agg(ys, srcg, dstg, ewpg, zeros_nd, out,
            src_r, dst_r, ewp_v, gbuf, sbuf0, sbuf1,
            acc, gsem, ssem0, ssem1, isem0, isem1, dsem0, dsem1):
    core = lax.axis_index("c")
    sid = lax.axis_index("s")
    wid = core * NS + sid

    pltpu.sync_copy(ewpg.at[wid], ewp_v)

    @pl.when(sid < NS - 1)
    def _():
        pltpu.sync_copy(zeros_nd.at[pl.ds(sid * RPT_A, RPT_A)],
                        acc.at[pl.ds(sid * RPT_A, RPT_A)])

    @pl.when(sid == NS - 1)
    def _():
        pltpu.sync_copy(zeros_nd.at[pl.ds((NS - 1) * RPT_A, RPT_B)],
                        acc.at[pl.ds((NS - 1) * RPT_A, RPT_B)])

    plsc.subcore_barrier()

    lanes = lax.iota(jnp.int32, NS)
    sbufs = (sbuf0, sbuf1)
    ssems = (ssem0, ssem1)
    isems = (isem0, isem1)
    dsems = (dsem0, dsem1)

    def scale(cur, sbuf):
        def group_body(g2, c2):
            ewp = ewp_v[cur, pl.ds(g2 * NS, NS)]
            we, wo = plsc.unpack(plsc.bitcast(ewp, jnp.bfloat16),
                                 format=plsc.PackFormat.INTERLEAVED)
            base = g2 * 2 * NS
            for rows, wv in ((base + 2 * lanes, we),
                             (base + 2 * lanes + 1, wo)):
                for w0 in range(0, H // 2, 8):
                    vs = [plsc.load_gather(
                              gbuf, [rows, jnp.full((NS,), w0 + k,
                                                    jnp.int32)])
                          for k in range(8)]
                    for k in range(8):
                        lo, hi = plsc.unpack(
                            plsc.bitcast(vs[k], jnp.bfloat16),
                            format=plsc.PackFormat.INTERLEAVED)
                        ca = jnp.full((NS,), 2 * (w0 + k), jnp.int32)
                        cb = jnp.full((NS,), 2 * (w0 + k) + 1, jnp.int32)
                        plsc.store_scatter(sbuf, [rows, ca], lo * wv)
                        plsc.store_scatter(sbuf, [rows, cb], hi * wv)
            return c2
        lax.fori_loop(0, K // (2 * NS), group_body, 0)

    pltpu.sync_copy(srcg.at[wid, 0], src_r.at[0])
    pltpu.sync_copy(srcg.at[wid, 1], src_r.at[1])
    pltpu.sync_copy(dstg.at[wid, 0], dst_r.at[0])
    pltpu.sync_copy(dstg.at[wid, 1], dst_r.at[1])
    pltpu.async_copy(srcg.at[wid, 2], src_r.at[2], isem0)
    pltpu.async_copy(srcg.at[wid, 3], src_r.at[3], isem1)
    pltpu.async_copy(dstg.at[wid, 2], dst_r.at[2], dsem0)
    pltpu.async_copy(dstg.at[wid, 3], dst_r.at[3], dsem1)
    pltpu.async_copy(ys.at[src_r.at[0]], gbuf, gsem)

    def pair_body(i, carry):
        for b in range(2):
            cur = 2 * i + b
            sbuf = sbufs[b]
            ssem = ssems[b]
            pltpu.make_async_copy(ys.at[src_r.at[cur % 4]], gbuf,
                                  gsem).wait()

            @pl.when(i > 0)
            def _():
                pltpu.make_async_copy(
                    sbuf, acc.at[dst_r.at[(cur - 2) % 4]], ssem).wait()

            scale(cur, sbuf)

            @pl.when(cur + 1 < NCHUNK)
            def _():
                @pl.when(cur >= 1)
                def _():
                    pltpu.make_async_copy(
                        srcg.at[wid, cur + 1],
                        src_r.at[(cur + 1) % 4], isems[1 - b]).wait()
                pltpu.async_copy(ys.at[src_r.at[(cur + 1) % 4]], gbuf,
                                 gsem)

            @pl.when(cur >= 2)
            def _():
                pltpu.make_async_copy(
                    dstg.at[wid, cur], dst_r.at[cur % 4], dsems[b]).wait()
            pltpu.async_copy(sbuf, acc.at[dst_r.at[cur % 4]], ssem,
                             add=True)

            @pl.when(cur + 4 < NCHUNK)
            def _():
                pltpu.async_copy(srcg.at[wid, cur + 4],
                                 src_r.at[(cur + 4) % 4], isems[b])
                pltpu.async_copy(dstg.at[wid, cur + 4],
                                 dst_r.at[(cur + 4) % 4], dsems[b])
        return carry
    lax.fori_loop(0, NCHUNK // 2, pair_body, 0)

    pltpu.make_async_copy(
        sbuf0, acc.at[dst_r.at[(NCHUNK - 2) % 4]], ssem0).wait()
    pltpu.make_async_copy(
        sbuf1, acc.at[dst_r.at[(NCHUNK - 1) % 4]], ssem1).wait()

    plsc.subcore_barrier()

    @pl.when(sid < NS - 1)
    def _():
        pltpu.sync_copy(acc.at[pl.ds(sid * RPT_A, RPT_A)],
                        out.at[core, pl.ds(sid * RPT_A, RPT_A)])

    @pl.when(sid == NS - 1)
    def _():
        pltpu.sync_copy(acc.at[pl.ds((NS - 1) * RPT_A, RPT_B)],
                        out.at[core, pl.ds((NS - 1) * RPT_A, RPT_B)])


def _tc_prep_body(x_ref, w_ref, wdeg_ref, ys_ref, dinv_ref):
    deg = 1.0 + jnp.sum(wdeg_ref[...], axis=1, keepdims=True)
    dinv = lax.rsqrt(deg)
    xw = jnp.dot(x_ref[...], w_ref[...],
                 preferred_element_type=jnp.float32,
                 precision=lax.Precision.HIGHEST)
    ys_ref[...] = xw * dinv
    dinv_ref[...] = dinv


def _tc_mid_body(acc_ref, ys_ref, dinv_ref, b_ref, w_ref, out_ref):
    dinv = dinv_ref[...]
    a = acc_ref[0] + acc_ref[1] + ys_ref[...]
    h = jnp.maximum(a * dinv + b_ref[...], 0.0)
    hw = jnp.dot(h, w_ref[...],
                 preferred_element_type=jnp.float32,
                 precision=lax.Precision.HIGHEST)
    out_ref[...] = hw * dinv


def _tc_final_body(acc_ref, ys_ref, dinv_ref, b_ref, batch_ref, wl_ref,
                   bl_ref, hg_ref, lp_ref):
    dinv = dinv_ref[...]
    h = (acc_ref[0] + acc_ref[1] + ys_ref[...]) * dinv + b_ref[...]
    seg = batch_ref[...]
    oh = (lax.broadcasted_iota(jnp.int32, (N, G), 1) == seg)
    hg = lax.dot_general(oh.astype(jnp.float32), h,
                         (((0,), (0,)), ((), ())),
                         preferred_element_type=jnp.float32,
                         precision=lax.Precision.HIGHEST)
    logits = jnp.dot(hg, wl_ref[...],
                     preferred_element_type=jnp.float32,
                     precision=lax.Precision.HIGHEST) + bl_ref[...]
    m = jnp.max(logits, axis=1, keepdims=True)
    lse = m + jnp.log(jnp.sum(jnp.exp(logits - m), axis=1, keepdims=True))
    hg_ref[...] = hg
    lp_ref[...] = logits - lse


_tc_prep = pl.pallas_call(
    _tc_prep_body,
    out_shape=(jax.ShapeDtypeStruct((N, H), jnp.float32),
               jax.ShapeDtypeStruct((N, 1), jnp.float32)),
)

_tc_mid = pl.pallas_call(
    _tc_mid_body,
    out_shape=jax.ShapeDtypeStruct((N, H), jnp.float32),
)

_tc_final = pl.pallas_call(
    _tc_final_body,
    out_shape=(jax.ShapeDtypeStruct((G, H), jnp.float32),
               jax.ShapeDtypeStruct((G, C), jnp.float32)),
)


def _pack_rows(ys):
    return jax.lax.bitcast_convert_type(
        ys.astype(jnp.bfloat16).reshape(N, H // 2, 2),
        jnp.int32).reshape(N, H // 2)


def kernel(x, edge_index, batch, edge_weight, W1, b1, W2, b2, W3, b3, Wl, bl):
    src = edge_index[0]
    dst = edge_index[1]
    pad = E_PAD - E
    i0 = jnp.zeros((pad,), jnp.int32)
    srcg = jnp.concatenate([src, i0]).reshape(NW, NCHUNK, K)
    dstg = jnp.concatenate([dst, i0]).reshape(NW, NCHUNK, K)
    ew_pad = jnp.concatenate([edge_weight,
                              jnp.zeros((pad,), edge_weight.dtype)])
    ewg = ew_pad.reshape(NW, NCHUNK, K)
    ewpg = jax.lax.bitcast_convert_type(
        ew_pad.astype(jnp.bfloat16).reshape(NW, NCHUNK, K // 2, 2),
        jnp.int32)
    zeros_nd = jnp.zeros((N, H), jnp.float32)
    b1r = b1.reshape(1, H)
    b2r = b2.reshape(1, H)
    b3r = b3.reshape(1, H)
    blr = bl.reshape(1, C)
    batchc = batch.reshape(N, 1)

    wdeg = _sc_wdeg(dstg, ewg).reshape(NW, N_PAD)[:, :N].T

    ys1, dinv = _tc_prep(x, W1, wdeg)
    acc1 = _sc_agg(_pack_rows(ys1), srcg, dstg, ewpg, zeros_nd)
    ys2 = _tc_mid(acc1, ys1, dinv, b1r, W2)
    acc2 = _sc_agg(_pack_rows(ys2), srcg, dstg, ewpg, zeros_nd)
    ys3 = _tc_mid(acc2, ys2, dinv, b2r, W3)
    acc3 = _sc_agg(_pack_rows(ys3), srcg, dstg, ewpg, zeros_nd)
    hG, logp = _tc_final(acc3, ys3, dinv, b3r, batchc, Wl, blr)
    return (hG, logp)


# speedup vs baseline: 1.2559x; 1.2559x over previous
import functools

import jax
import jax.numpy as jnp
from jax import lax
from jax.experimental import pallas as pl
from jax.experimental.pallas import tpu as pltpu
from jax.experimental.pallas import tpu_sc as plsc

N = 10000
E = 320000
D = 128
H = 128
C = 10
G = 64

NC = 2
NS = 16
NW = NC * NS
K = 128
NCHUNK = 80
E_PAD = NW * K * NCHUNK
N_PAD = 10240
RPT_A = 632
RPT_B = N - (NS - 1) * RPT_A

_mesh = plsc.VectorSubcoreMesh(core_axis_name="c", subcore_axis_name="s")


@functools.partial(
    pl.kernel,
    out_type=jax.ShapeDtypeStruct((NW * N_PAD,), jnp.float32),
    mesh=_mesh,
    scratch_types=[
        pltpu.VMEM((NCHUNK, K), jnp.int32),
        pltpu.VMEM((NCHUNK, K), jnp.float32),
        pltpu.VMEM((N_PAD,), jnp.float32),
    ],
    compiler_params=pltpu.CompilerParams(needs_layout_passes=False),
)
def _sc_wdeg(dstg, ewg, out, dst_v, ew_v, hist):
    core = lax.axis_index("c")
    sid = lax.axis_index("s")
    wid = core * NS + sid

    pltpu.sync_copy(dstg.at[wid], dst_v)
    pltpu.sync_copy(ewg.at[wid], ew_v)

    zv = jnp.zeros((NS,), jnp.float32)

    def zero_body(i, carry):
        hist[pl.ds(i * NS, NS)] = zv
        return carry
    lax.fori_loop(0, N_PAD // NS, zero_body, 0)

    def chunk_body(j, carry):
        def group_body(g, c2):
            idx = dst_v[j, pl.ds(g * NS, NS)]
            w = ew_v[j, pl.ds(g * NS, NS)]
            plsc.addupdate_scatter(hist, [idx], w)
            return c2
        lax.fori_loop(0, K // NS, group_body, 0)
        return carry
    lax.fori_loop(0, NCHUNK, chunk_body, 0)

    pltpu.sync_copy(hist, out.at[pl.ds(wid * N_PAD, N_PAD)])


@functools.partial(
    pl.kernel,
    out_type=jax.ShapeDtypeStruct((NC, N, H), jnp.float32),
    mesh=_mesh,
    scratch_types=[
        pltpu.VMEM((4, K), jnp.int32),
        pltpu.VMEM((4, K), jnp.int32),
        pltpu.VMEM((4, K // 2), jnp.int32),
        pltpu.VMEM((K, H // 2), jnp.int32),
        pltpu.VMEM((K, H // 2), jnp.int32),
        pltpu.VMEM((K, H), jnp.float32),
        pltpu.VMEM((K, H), jnp.float32),
        pltpu.VMEM_SHARED((N, H), jnp.float32),
        pltpu.SemaphoreType.DMA,
        pltpu.SemaphoreType.DMA,
        pltpu.SemaphoreType.DMA,
        pltpu.SemaphoreType.DMA,
        pltpu.SemaphoreType.DMA,
        pltpu.SemaphoreType.DMA,
        pltpu.SemaphoreType.DMA,
        pltpu.SemaphoreType.DMA,
        pltpu.SemaphoreType.DMA,
        pltpu.SemaphoreType.DMA,
    ],
    compiler_params=pltpu.CompilerParams(needs_layout_passes=False,
                                         use_tc_tiling_on_sc=False),
)
def _sc_agg(ys, srcg, dstg, ewpg, zeros_nd, out,
            src_r, dst_r, ewp_r, gbuf0, gbuf1, sbuf0, sbuf1,
            acc, gsem0, gsem1, ssem0, ssem1, isem0, isem1, dsem0, dsem1,
            esem0, esem1):
    core = lax.axis_index("c")
    sid = lax.axis_index("s")
    wid = core * NS + sid

    @pl.when(sid < NS - 1)
    def _():
        pltpu.sync_copy(zeros_nd.at[pl.ds(sid * RPT_A, RPT_A)],
                        acc.at[pl.ds(sid * RPT_A, RPT_A)])

    @pl.when(sid == NS - 1)
    def _():
        pltpu.sync_copy(zeros_nd.at[pl.ds((NS - 1) * RPT_A, RPT_B)],
                        acc.at[pl.ds((NS - 1) * RPT_A, RPT_B)])

    plsc.subcore_barrier()

    lanes = lax.iota(jnp.int32, NS)
    gbufs = (gbuf0, gbuf1)
    gsems = (gsem0, gsem1)
    sbufs = (sbuf0, sbuf1)
    ssems = (ssem0, ssem1)
    isems = (isem0, isem1)
    dsems = (dsem0, dsem1)
    esems = (esem0, esem1)

    def scale(cur, gbuf, sbuf):
        def group_body(g2, c2):
            ewp = ewp_r[cur % 4, pl.ds(g2 * NS, NS)]
            we, wo = plsc.unpack(plsc.bitcast(ewp, jnp.bfloat16),
                                 format=plsc.PackFormat.INTERLEAVED)
            base = g2 * 2 * NS
            for rows, wv in ((base + 2 * lanes, we),
                             (base + 2 * lanes + 1, wo)):
                for w0 in range(0, H // 2, 8):
                    vs = [plsc.load_gather(
                              gbuf, [rows, jnp.full((NS,), w0 + k,
                                                    jnp.int32)])
                          for k in range(8)]
                    for k in range(8):
                        lo, hi = plsc.unpack(
                            plsc.bitcast(vs[k], jnp.bfloat16),
                            format=plsc.PackFormat.INTERLEAVED)
                        ca = jnp.full((NS,), 2 * (w0 + k), jnp.int32)
                        cb = jnp.full((NS,), 2 * (w0 + k) + 1, jnp.int32)
                        plsc.store_scatter(sbuf, [rows, ca], lo * wv)
                        plsc.store_scatter(sbuf, [rows, cb], hi * wv)
            return c2
        lax.fori_loop(0, K // (2 * NS), group_body, 0)

    pltpu.sync_copy(srcg.at[wid, 0], src_r.at[0])
    pltpu.sync_copy(srcg.at[wid, 1], src_r.at[1])
    pltpu.sync_copy(dstg.at[wid, 0], dst_r.at[0])
    pltpu.sync_copy(dstg.at[wid, 1], dst_r.at[1])
    pltpu.sync_copy(ewpg.at[wid, 0], ewp_r.at[0])
    pltpu.sync_copy(ewpg.at[wid, 1], ewp_r.at[1])
    pltpu.async_copy(srcg.at[wid, 2], src_r.at[2], isem0)
    pltpu.async_copy(srcg.at[wid, 3], src_r.at[3], isem1)
    pltpu.async_copy(dstg.at[wid, 2], dst_r.at[2], dsem0)
    pltpu.async_copy(dstg.at[wid, 3], dst_r.at[3], dsem1)
    pltpu.async_copy(ewpg.at[wid, 2], ewp_r.at[2], esem0)
    pltpu.async_copy(ewpg.at[wid, 3], ewp_r.at[3], esem1)
    pltpu.async_copy(ys.at[src_r.at[0]], gbuf0, gsem0)
    pltpu.async_copy(ys.at[src_r.at[1]], gbuf1, gsem1)

    def pair_body(i, carry):
        for b in range(2):
            cur = 2 * i + b
            gbuf, sbuf = gbufs[b], sbufs[b]
            gsem, ssem = gsems[b], ssems[b]
            pltpu.make_async_copy(ys.at[src_r.at[cur % 4]], gbuf,
                                  gsem).wait()

            @pl.when(i > 0)
            def _():
                pltpu.make_async_copy(
                    sbuf, acc.at[dst_r.at[(cur - 2) % 4]], ssem).wait()

            @pl.when(cur >= 2)
            def _():
                pltpu.make_async_copy(
                    ewpg.at[wid, cur], ewp_r.at[cur % 4], esems[b]).wait()

            scale(cur, gbuf, sbuf)

            @pl.when(cur + 2 < NCHUNK)
            def _():
                pltpu.make_async_copy(
                    srcg.at[wid, cur + 2],
                    src_r.at[(cur + 2) % 4], isems[b]).wait()
                pltpu.async_copy(ys.at[src_r.at[(cur + 2) % 4]], gbuf,
                                 gsem)

            @pl.when(cur >= 2)
            def _():
                pltpu.make_async_copy(
                    dstg.at[wid, cur], dst_r.at[cur % 4], dsems[b]).wait()
            pltpu.async_copy(sbuf, acc.at[dst_r.at[cur % 4]], ssem,
                             add=True)

            @pl.when(cur + 4 < NCHUNK)
            def _():
                pltpu.async_copy(srcg.at[wid, cur + 4],
                                 src_r.at[(cur + 4) % 4], isems[b])
                pltpu.async_copy(dstg.at[wid, cur + 4],
                                 dst_r.at[(cur + 4) % 4], dsems[b])
                pltpu.async_copy(ewpg.at[wid, cur + 4],
                                 ewp_r.at[(cur + 4) % 4], esems[b])
        return carry
    lax.fori_loop(0, NCHUNK // 2, pair_body, 0)

    pltpu.make_async_copy(
        sbuf0, acc.at[dst_r.at[(NCHUNK - 2) % 4]], ssem0).wait()
    pltpu.make_async_copy(
        sbuf1, acc.at[dst_r.at[(NCHUNK - 1) % 4]], ssem1).wait()

    plsc.subcore_barrier()

    @pl.when(sid < NS - 1)
    def _():
        pltpu.sync_copy(acc.at[pl.ds(sid * RPT_A, RPT_A)],
                        out.at[core, pl.ds(sid * RPT_A, RPT_A)])

    @pl.when(sid == NS - 1)
    def _():
        pltpu.sync_copy(acc.at[pl.ds((NS - 1) * RPT_A, RPT_B)],
                        out.at[core, pl.ds((NS - 1) * RPT_A, RPT_B)])


def _tc_prep_body(x_ref, w_ref, wdeg_ref, ys_ref, dinv_ref):
    deg = 1.0 + jnp.sum(wdeg_ref[...], axis=1, keepdims=True)
    dinv = lax.rsqrt(deg)
    xw = jnp.dot(x_ref[...], w_ref[...],
                 preferred_element_type=jnp.float32,
                 precision=lax.Precision.HIGHEST)
    ys_ref[...] = xw * dinv
    dinv_ref[...] = dinv


def _tc_mid_body(acc_ref, ys_ref, dinv_ref, b_ref, w_ref, out_ref):
    dinv = dinv_ref[...]
    a = acc_ref[0] + acc_ref[1] + ys_ref[...]
    h = jnp.maximum(a * dinv + b_ref[...], 0.0)
    hw = jnp.dot(h, w_ref[...],
                 preferred_element_type=jnp.float32,
                 precision=lax.Precision.HIGHEST)
    out_ref[...] = hw * dinv


def _tc_final_body(acc_ref, ys_ref, dinv_ref, b_ref, batch_ref, wl_ref,
                   bl_ref, hg_ref, lp_ref):
    dinv = dinv_ref[...]
    h = (acc_ref[0] + acc_ref[1] + ys_ref[...]) * dinv + b_ref[...]
    seg = batch_ref[...]
    oh = (lax.broadcasted_iota(jnp.int32, (N, G), 1) == seg)
    hg = lax.dot_general(oh.astype(jnp.float32), h,
                         (((0,), (0,)), ((), ())),
                         preferred_element_type=jnp.float32,
                         precision=lax.Precision.HIGHEST)
    logits = jnp.dot(hg, wl_ref[...],
                     preferred_element_type=jnp.float32,
                     precision=lax.Precision.HIGHEST) + bl_ref[...]
    m = jnp.max(logits, axis=1, keepdims=True)
    lse = m + jnp.log(jnp.sum(jnp.exp(logits - m), axis=1, keepdims=True))
    hg_ref[...] = hg
    lp_ref[...] = logits - lse


_tc_prep = pl.pallas_call(
    _tc_prep_body,
    out_shape=(jax.ShapeDtypeStruct((N, H), jnp.float32),
               jax.ShapeDtypeStruct((N, 1), jnp.float32)),
)

_tc_mid = pl.pallas_call(
    _tc_mid_body,
    out_shape=jax.ShapeDtypeStruct((N, H), jnp.float32),
)

_tc_final = pl.pallas_call(
    _tc_final_body,
    out_shape=(jax.ShapeDtypeStruct((G, H), jnp.float32),
               jax.ShapeDtypeStruct((G, C), jnp.float32)),
)


def _pack_rows(ys):
    return jax.lax.bitcast_convert_type(
        ys.astype(jnp.bfloat16).reshape(N, H // 2, 2),
        jnp.int32).reshape(N, H // 2)


def kernel(x, edge_index, batch, edge_weight, W1, b1, W2, b2, W3, b3, Wl, bl):
    src = edge_index[0]
    dst = edge_index[1]
    pad = E_PAD - E
    i0 = jnp.zeros((pad,), jnp.int32)
    srcg = jnp.concatenate([src, i0]).reshape(NW, NCHUNK, K)
    dstg = jnp.concatenate([dst, i0]).reshape(NW, NCHUNK, K)
    ew_pad = jnp.concatenate([edge_weight,
                              jnp.zeros((pad,), edge_weight.dtype)])
    ewg = ew_pad.reshape(NW, NCHUNK, K)
    ewpg = jax.lax.bitcast_convert_type(
        ew_pad.astype(jnp.bfloat16).reshape(NW, NCHUNK, K // 2, 2),
        jnp.int32)
    zeros_nd = jnp.zeros((N, H), jnp.float32)
    b1r = b1.reshape(1, H)
    b2r = b2.reshape(1, H)
    b3r = b3.reshape(1, H)
    blr = bl.reshape(1, C)
    batchc = batch.reshape(N, 1)

    wdeg = _sc_wdeg(dstg, ewg).reshape(NW, N_PAD)[:, :N].T

    ys1, dinv = _tc_prep(x, W1, wdeg)
    acc1 = _sc_agg(_pack_rows(ys1), srcg, dstg, ewpg, zeros_nd)
    ys2 = _tc_mid(acc1, ys1, dinv, b1r, W2)
    acc2 = _sc_agg(_pack_rows(ys2), srcg, dstg, ewpg, zeros_nd)
    ys3 = _tc_mid(acc2, ys2, dinv, b2r, W3)
    acc3 = _sc_agg(_pack_rows(ys3), srcg, dstg, ewpg, zeros_nd)
    hG, logp = _tc_final(acc3, ys3, dinv, b3r, batchc, Wl, blr)
    return (hG, logp)
